# baseline (speedup 1.0000x reference)
import functools

import jax
import jax.numpy as jnp
from jax import lax
from jax.experimental import pallas as pl
from jax.experimental.pallas import tpu as pltpu
from jax.experimental.pallas import tpu_sc as plsc

_N = 10000
_NP = 10240
_NC = 2
_NS = 16
_K = 128
_G = 4
_ROWS_PER_SUB = _NP // _NS

_BN_EPS = 1e-5
_SELU_SCALE = 1.0507009873554805
_SELU_ALPHA = 1.6732632423543772


def _make_prop(fh, e_pad, n_steps, gsz):
    e_per_worker = e_pad // _NS
    rows_per_worker = e_per_worker // _K
    n_groups = rows_per_worker // gsz
    assert rows_per_worker % gsz == 0 and n_groups % 2 == 0
    bank_rows = gsz * _K

    mesh = plsc.VectorSubcoreMesh(core_axis_name="c", subcore_axis_name="s",
                                  num_cores=_NC, num_subcores=_NS)

    @functools.partial(
        pl.kernel,
        out_type=jax.ShapeDtypeStruct((_NC, _NP, fh), jnp.float32),
        mesh=mesh,
        scratch_types=[
            pltpu.VMEM_SHARED((_NP, fh), jnp.float32),
            pltpu.VMEM_SHARED((_NP, fh), jnp.float32),
            pltpu.VMEM((rows_per_worker, _K), jnp.int32),
            pltpu.VMEM((rows_per_worker, _K), jnp.int32),
            pltpu.VMEM((bank_rows, fh), jnp.float32),
            pltpu.VMEM((bank_rows, fh), jnp.float32),
            pltpu.SemaphoreType.DMA,
            pltpu.SemaphoreType.DMA,
            pltpu.SemaphoreType.DMA,
            pltpu.SemaphoreType.DMA,
            pltpu.SemaphoreType.DMA,
        ],
        compiler_params=pltpu.CompilerParams(use_tc_tiling_on_sc=False),
    )
    def prop(h_hbm, src_hbm, dst_hbm, out_hbm, acc, h_st, src_v, dst_v,
             rows0, rows1, sem0, semg0, semg1, sems0, sems1):
        c = lax.axis_index("c")
        s = lax.axis_index("s")
        row0 = s * _ROWS_PER_SUB
        base_row = s * rows_per_worker
        nsl = pl.ds(row0, _ROWS_PER_SUB)

        pltpu.async_copy(h_hbm.at[c].at[nsl], h_st.at[nsl], sem0)
        pltpu.async_copy(h_hbm.at[c].at[nsl], acc.at[nsl], sem0)
        pltpu.async_copy(src_hbm.at[pl.ds(base_row, rows_per_worker)], src_v,
                         sem0)
        pltpu.async_copy(dst_hbm.at[pl.ds(base_row, rows_per_worker)], dst_v,
                         sem0)
        pltpu.make_async_copy(h_hbm.at[c].at[nsl], h_st.at[nsl], sem0).wait()
        pltpu.make_async_copy(h_hbm.at[c].at[nsl], acc.at[nsl], sem0).wait()
        pltpu.make_async_copy(
            src_hbm.at[pl.ds(base_row, rows_per_worker)], src_v, sem0).wait()
        pltpu.make_async_copy(
            dst_hbm.at[pl.ds(base_row, rows_per_worker)], dst_v, sem0).wait()
        plsc.subcore_barrier()

        banks = ((rows0, semg0, sems0), (rows1, semg1, sems1))

        def gather_desc(g, rows_b, semg, j):
            return pltpu.make_async_copy(h_st.at[src_v.at[g * gsz + j]],
                                         rows_b.at[pl.ds(j * _K, _K)], semg)

        def scatter_desc(g, rows_b, sems, j):
            return pltpu.make_async_copy(rows_b.at[pl.ds(j * _K, _K)],
                                         acc.at[dst_v.at[g * gsz + j]], sems)

        def edge_pass():
            rows_p, semg_p, _ = banks[0]
            for j in range(gsz):
                gather_desc(0, rows_p, semg_p, j).start()

            @pl.loop(0, n_groups, step=2)
            def _(g0):
                for bi in range(2):
                    rows_c, semg_c, sems_c = banks[bi]
                    rows_n, semg_n, sems_n = banks[1 - bi]
                    g = g0 + bi

                    @pl.when(g >= 1)
                    def _():
                        for j in range(gsz):
                            scatter_desc(g - 1, rows_n, sems_n, j).wait()

                    @pl.when(g + 1 < n_groups)
                    def _():
                        for j in range(gsz):
                            gather_desc(g + 1, rows_n, semg_n, j).start()

                    for j in range(gsz):
                        gather_desc(g, rows_c, semg_c, j).wait()
                    for j in range(gsz):
                        scatter_desc(g, rows_c, sems_c, j).start(add=True)

            g_last = n_groups - 1
            rows_b, _, sems = banks[g_last % 2]
            for j in range(gsz):
                scatter_desc(g_last, rows_b, sems, j).wait()

        edge_pass()
        for _step in range(n_steps - 1):
            plsc.subcore_barrier()
            pieces = []
            off = 0
            while off < _ROWS_PER_SUB:
                sz = min(bank_rows, _ROWS_PER_SUB - off)
                pieces.append((off, sz, banks[len(pieces) % 2]))
                off += sz
            for off, sz, (rows_b, semg, _) in pieces:
                pltpu.async_copy(acc.at[pl.ds(row0 + off, sz)],
                                 rows_b.at[pl.ds(0, sz)], semg)
            for off, sz, (rows_b, semg, _) in pieces:
                pltpu.make_async_copy(acc.at[pl.ds(row0 + off, sz)],
                                      rows_b.at[pl.ds(0, sz)], semg).wait()
                pltpu.sync_copy(rows_b.at[pl.ds(0, sz)],
                                h_st.at[pl.ds(row0 + off, sz)])
            plsc.subcore_barrier()
            edge_pass()

        plsc.subcore_barrier()
        pltpu.sync_copy(acc.at[nsl], out_hbm.at[c].at[nsl])

    return prop


def _mm1(x, w1, ei, e_pad):
    n = x.shape[0]
    fh = w1.shape[1] // 2
    e_rows = ei.shape[1]
    pad_rows = e_pad // _K - e_rows

    def body(x_ref, w_ref, e_ref, o_ref, src_ref, dst_ref):
        y = jnp.dot(x_ref[...], w_ref[...], preferred_element_type=jnp.float32)
        zrows = jnp.zeros((_NP - n, fh), jnp.float32)
        o_ref[0] = jnp.concatenate([y[:, :fh], zrows], axis=0)
        o_ref[1] = jnp.concatenate([y[:, fh:], zrows], axis=0)
        src_ref[...] = jnp.concatenate(
            [e_ref[0], jnp.zeros((pad_rows, _K), jnp.int32)], axis=0)
        flat = (lax.broadcasted_iota(jnp.int32, (pad_rows, _K), 0) * _K
                + lax.broadcasted_iota(jnp.int32, (pad_rows, _K), 1))
        dump = _N + lax.rem(flat, jnp.int32(_NP - _N))
        dst_ref[...] = jnp.concatenate([e_ref[1], dump], axis=0)

    return pl.pallas_call(
        body,
        out_shape=(
            jax.ShapeDtypeStruct((2, _NP, fh), jnp.float32),
            jax.ShapeDtypeStruct((e_pad // _K, _K), jnp.int32),
            jax.ShapeDtypeStruct((e_pad // _K, _K), jnp.int32),
        ),
    )(x, w1, ei)


def _mid(p, gamma, beta, w2):
    npad = p.shape[1]
    ch = w2.shape[1] // 2

    def body(p_ref, g_ref, b_ref, w_ref, o_ref):
        z = jnp.concatenate([p_ref[0], p_ref[1]], axis=1)
        rows = lax.broadcasted_iota(jnp.int32, (npad, 1), 0)
        mask = (rows < _N).astype(jnp.float32)
        zm = z * mask
        mean = jnp.sum(zm, axis=0, keepdims=True) * (1.0 / _N)
        d = (z - mean) * mask
        var = jnp.sum(d * d, axis=0, keepdims=True) * (1.0 / _N)
        hn = (z - mean) * jax.lax.rsqrt(var + _BN_EPS) * g_ref[...] + b_ref[...]
        hs = _SELU_SCALE * jnp.where(hn > 0, hn,
                                     _SELU_ALPHA * (jnp.exp(hn) - 1.0))
        y = jnp.dot(hs, w_ref[...], preferred_element_type=jnp.float32)
        o_ref[0] = y[:, :ch]
        o_ref[1] = y[:, ch:]

    return pl.pallas_call(
        body,
        out_shape=jax.ShapeDtypeStruct((2, npad, ch), jnp.float32),
    )(p, gamma, beta, w2)


def _final(p, b2):
    npad = p.shape[1]
    cdim = 2 * p.shape[2]

    def body(p_ref, b_ref, o_ref):
        z = jnp.concatenate([p_ref[0], p_ref[1]], axis=1) + b_ref[...]
        m = jnp.max(z, axis=1, keepdims=True)
        lse = jnp.log(jnp.sum(jnp.exp(z - m), axis=1, keepdims=True)) + m
        o_ref[...] = z - lse

    return pl.pallas_call(
        body,
        out_shape=jax.ShapeDtypeStruct((npad, cdim), jnp.float32),
    )(p, b2)


@jax.jit
def kernel(x, edge_index, W1, b1, gamma, beta, W2, b2):
    del b1
    n, _ = x.shape
    e = edge_index.shape[1]
    quantum = _NS * _K * _G
    e_pad = ((e + quantum - 1) // quantum) * quantum

    ei = edge_index.astype(jnp.int32).reshape(2, e // _K, _K)

    prop_h = _make_prop(W1.shape[1] // 2, e_pad, 2, _G)
    prop_c = _make_prop(W2.shape[1] // 2, e_pad, 1, _G)

    y1, src_p, dst_p = _mm1(x, W1, ei, e_pad)
    p2 = prop_h(y1, src_p, dst_p)
    y2 = _mid(p2, gamma.reshape(1, -1), beta.reshape(1, -1), W2)
    p3 = prop_c(y2, src_p, dst_p)
    out = _final(p3, b2.reshape(1, -1))
    return out[:n]


# speedup vs baseline: 1.0227x; 1.0227x over previous
import functools

import jax
import jax.numpy as jnp
from jax import lax
from jax.experimental import pallas as pl
from jax.experimental.pallas import tpu as pltpu
from jax.experimental.pallas import tpu_sc as plsc

_N = 10000
_NP = 10240
_NC = 2
_NS = 16
_K = 128
_G = 4
_ROWS_PER_SUB = _NP // _NS

_BN_EPS = 1e-5
_SELU_SCALE = 1.0507009873554805
_SELU_ALPHA = 1.6732632423543772


def _make_prop(fh, e_pad, n_steps, gsz):
    e_per_worker = e_pad // _NS
    rows_per_worker = e_per_worker // _K
    n_groups = rows_per_worker // gsz
    assert rows_per_worker % gsz == 0 and n_groups % 2 == 0
    bank_rows = gsz * _K

    mesh = plsc.VectorSubcoreMesh(core_axis_name="c", subcore_axis_name="s",
                                  num_cores=_NC, num_subcores=_NS)

    @functools.partial(
        pl.kernel,
        out_type=jax.ShapeDtypeStruct((_NC, _NP, fh), jnp.float32),
        mesh=mesh,
        scratch_types=[
            pltpu.VMEM_SHARED((_NP, fh), jnp.float32),
            pltpu.VMEM_SHARED((_NP, fh), jnp.float32),
            pltpu.VMEM((rows_per_worker, _K), jnp.int32),
            pltpu.VMEM((rows_per_worker, _K), jnp.int32),
            pltpu.VMEM((bank_rows, fh), jnp.float32),
            pltpu.VMEM((bank_rows, fh), jnp.float32),
            pltpu.SemaphoreType.DMA,
            pltpu.SemaphoreType.DMA,
            pltpu.SemaphoreType.DMA,
            pltpu.SemaphoreType.DMA,
            pltpu.SemaphoreType.DMA,
        ],
        compiler_params=pltpu.CompilerParams(use_tc_tiling_on_sc=False),
    )
    def prop(h_hbm, src_hbm, dst_hbm, out_hbm, acc, h_st, src_v, dst_v,
             rows0, rows1, sem0, semg0, semg1, sems0, sems1):
        c = lax.axis_index("c")
        s = lax.axis_index("s")
        row0 = s * _ROWS_PER_SUB
        base_row = s * rows_per_worker
        nsl = pl.ds(row0, _ROWS_PER_SUB)

        pltpu.async_copy(h_hbm.at[c].at[nsl], h_st.at[nsl], sem0)
        pltpu.async_copy(h_hbm.at[c].at[nsl], acc.at[nsl], sem0)
        pltpu.async_copy(src_hbm.at[pl.ds(base_row, rows_per_worker)], src_v,
                         sem0)
        pltpu.async_copy(dst_hbm.at[pl.ds(base_row, rows_per_worker)], dst_v,
                         sem0)
        pltpu.make_async_copy(h_hbm.at[c].at[nsl], h_st.at[nsl], sem0).wait()
        pltpu.make_async_copy(h_hbm.at[c].at[nsl], acc.at[nsl], sem0).wait()
        pltpu.make_async_copy(
            src_hbm.at[pl.ds(base_row, rows_per_worker)], src_v, sem0).wait()
        pltpu.make_async_copy(
            dst_hbm.at[pl.ds(base_row, rows_per_worker)], dst_v, sem0).wait()
        plsc.subcore_barrier()

        banks = ((rows0, semg0, sems0), (rows1, semg1, sems1))

        def gather_desc(g, rows_b, semg, j):
            return pltpu.make_async_copy(h_st.at[src_v.at[g * gsz + j]],
                                         rows_b.at[pl.ds(j * _K, _K)], semg)

        def scatter_desc(g, rows_b, sems, j):
            return pltpu.make_async_copy(rows_b.at[pl.ds(j * _K, _K)],
                                         acc.at[dst_v.at[g * gsz + j]], sems)

        def edge_pass():
            rows_p, semg_p, _ = banks[0]
            for j in range(gsz):
                gather_desc(0, rows_p, semg_p, j).start()

            @pl.loop(0, n_groups, step=2)
            def _(g0):
                for bi in range(2):
                    rows_c, semg_c, sems_c = banks[bi]
                    rows_n, semg_n, sems_n = banks[1 - bi]
                    g = g0 + bi

                    @pl.when(g >= 1)
                    def _():
                        for j in range(gsz):
                            scatter_desc(g - 1, rows_n, sems_n, j).wait()

                    @pl.when(g + 1 < n_groups)
                    def _():
                        for j in range(gsz):
                            gather_desc(g + 1, rows_n, semg_n, j).start()

                    for j in range(gsz):
                        gather_desc(g, rows_c, semg_c, j).wait()
                    for j in range(gsz):
                        scatter_desc(g, rows_c, sems_c, j).start(add=True)

            g_last = n_groups - 1
            rows_b, _, sems = banks[g_last % 2]
            for j in range(gsz):
                scatter_desc(g_last, rows_b, sems, j).wait()

        edge_pass()
        for _step in range(n_steps - 1):
            plsc.subcore_barrier()
            pieces = []
            off = 0
            while off < _ROWS_PER_SUB:
                sz = min(bank_rows, _ROWS_PER_SUB - off)
                pieces.append((off, sz, banks[len(pieces) % 2]))
                off += sz
            for off, sz, (rows_b, semg, _) in pieces:
                pltpu.async_copy(acc.at[pl.ds(row0 + off, sz)],
                                 rows_b.at[pl.ds(0, sz)], semg)
            for off, sz, (rows_b, semg, _) in pieces:
                pltpu.make_async_copy(acc.at[pl.ds(row0 + off, sz)],
                                      rows_b.at[pl.ds(0, sz)], semg).wait()
                pltpu.sync_copy(rows_b.at[pl.ds(0, sz)],
                                h_st.at[pl.ds(row0 + off, sz)])
            plsc.subcore_barrier()
            edge_pass()

        plsc.subcore_barrier()
        pltpu.sync_copy(acc.at[nsl], out_hbm.at[c].at[nsl])

    return prop


def _mm1(x, w1, ei, e_pad):
    n = x.shape[0]
    fh = w1.shape[1] // 2
    e_rows = ei.shape[1] // _K
    pad_rows = e_pad // _K - e_rows

    def body(x_ref, w_ref, e_ref, o_ref, src_ref, dst_ref):
        y = jnp.dot(x_ref[...], w_ref[...], preferred_element_type=jnp.float32)
        zrows = jnp.zeros((_NP - n, fh), jnp.float32)
        o_ref[0] = jnp.concatenate([y[:, :fh], zrows], axis=0)
        o_ref[1] = jnp.concatenate([y[:, fh:], zrows], axis=0)
        src_ref[...] = jnp.concatenate(
            [jnp.reshape(e_ref[0], (e_rows, _K)),
             jnp.zeros((pad_rows, _K), jnp.int32)], axis=0)
        flat = (lax.broadcasted_iota(jnp.int32, (pad_rows, _K), 0) * _K
                + lax.broadcasted_iota(jnp.int32, (pad_rows, _K), 1))
        dump = _N + lax.rem(flat, jnp.int32(_NP - _N))
        dst_ref[...] = jnp.concatenate(
            [jnp.reshape(e_ref[1], (e_rows, _K)), dump], axis=0)

    return pl.pallas_call(
        body,
        out_shape=(
            jax.ShapeDtypeStruct((2, _NP, fh), jnp.float32),
            jax.ShapeDtypeStruct((e_pad // _K, _K), jnp.int32),
            jax.ShapeDtypeStruct((e_pad // _K, _K), jnp.int32),
        ),
    )(x, w1, ei)


def _mid(p, gamma, beta, w2):
    npad = p.shape[1]
    ch = w2.shape[1] // 2

    def body(p_ref, g_ref, b_ref, w_ref, o_ref):
        z = jnp.concatenate([p_ref[0], p_ref[1]], axis=1)
        rows = lax.broadcasted_iota(jnp.int32, (npad, 1), 0)
        mask = (rows < _N).astype(jnp.float32)
        zm = z * mask
        mean = jnp.sum(zm, axis=0, keepdims=True) * (1.0 / _N)
        d = (z - mean) * mask
        var = jnp.sum(d * d, axis=0, keepdims=True) * (1.0 / _N)
        hn = (z - mean) * jax.lax.rsqrt(var + _BN_EPS) * g_ref[...] + b_ref[...]
        hs = _SELU_SCALE * jnp.where(hn > 0, hn,
                                     _SELU_ALPHA * (jnp.exp(hn) - 1.0))
        y = jnp.dot(hs, w_ref[...], preferred_element_type=jnp.float32)
        o_ref[0] = y[:, :ch]
        o_ref[1] = y[:, ch:]

    return pl.pallas_call(
        body,
        out_shape=jax.ShapeDtypeStruct((2, npad, ch), jnp.float32),
    )(p, gamma, beta, w2)


def _final(p, b2, n):
    cdim = 2 * p.shape[2]

    def body(p_ref, b_ref, o_ref):
        z = jnp.concatenate([p_ref[0, :n], p_ref[1, :n]], axis=1) + b_ref[...]
        m = jnp.max(z, axis=1, keepdims=True)
        lse = jnp.log(jnp.sum(jnp.exp(z - m), axis=1, keepdims=True)) + m
        o_ref[...] = z - lse

    return pl.pallas_call(
        body,
        out_shape=jax.ShapeDtypeStruct((n, cdim), jnp.float32),
    )(p, b2)


@jax.jit
def kernel(x, edge_index, W1, b1, gamma, beta, W2, b2):
    del b1
    n, _ = x.shape
    e = edge_index.shape[1]
    quantum = _NS * _K * _G
    e_pad = ((e + quantum - 1) // quantum) * quantum

    ei = edge_index.astype(jnp.int32)

    prop_h = _make_prop(W1.shape[1] // 2, e_pad, 2, _G)
    prop_c = _make_prop(W2.shape[1] // 2, e_pad, 1, _G)

    y1, src_p, dst_p = _mm1(x, W1, ei, e_pad)
    p2 = prop_h(y1, src_p, dst_p)
    y2 = _mid(p2, gamma.reshape(1, -1), beta.reshape(1, -1), W2)
    p3 = prop_c(y2, src_p, dst_p)
    return _final(p3, b2.reshape(1, -1), n)


# speedup vs baseline: 1.0447x; 1.0215x over previous
import functools

import jax
import jax.numpy as jnp
from jax import lax
from jax.experimental import pallas as pl
from jax.experimental.pallas import tpu as pltpu
from jax.experimental.pallas import tpu_sc as plsc

_N = 10000
_NP = 10240
_NC = 2
_NS = 16
_K = 128
_G = 4
_ROWS_PER_SUB = _NP // _NS

_BN_EPS = 1e-5
_SELU_SCALE = 1.0507009873554805
_SELU_ALPHA = 1.6732632423543772


def _make_prop(fh, e_pad, n_steps, gsz):
    e_per_worker = e_pad // _NS
    rows_per_worker = e_per_worker // _K
    n_groups = rows_per_worker // gsz
    assert rows_per_worker % gsz == 0 and n_groups % 2 == 0
    bank_rows = gsz * _K

    mesh = plsc.VectorSubcoreMesh(core_axis_name="c", subcore_axis_name="s",
                                  num_cores=_NC, num_subcores=_NS)

    @functools.partial(
        pl.kernel,
        out_type=jax.ShapeDtypeStruct((_NP, _NC * fh), jnp.float32),
        mesh=mesh,
        scratch_types=[
            pltpu.VMEM_SHARED((_NP, fh), jnp.float32),
            pltpu.VMEM_SHARED((_NP, fh), jnp.float32),
            pltpu.VMEM((rows_per_worker, _K), jnp.int32),
            pltpu.VMEM((rows_per_worker, _K), jnp.int32),
            pltpu.VMEM((bank_rows, fh), jnp.float32),
            pltpu.VMEM((bank_rows, fh), jnp.float32),
            pltpu.SemaphoreType.DMA,
            pltpu.SemaphoreType.DMA,
            pltpu.SemaphoreType.DMA,
            pltpu.SemaphoreType.DMA,
            pltpu.SemaphoreType.DMA,
        ],
        compiler_params=pltpu.CompilerParams(use_tc_tiling_on_sc=False),
    )
    def prop(h_hbm, src_hbm, dst_hbm, out_hbm, acc, h_st, src_v, dst_v,
             rows0, rows1, sem0, semg0, semg1, sems0, sems1):
        c = lax.axis_index("c")
        s = lax.axis_index("s")
        row0 = s * _ROWS_PER_SUB
        base_row = s * rows_per_worker
        nsl = pl.ds(row0, _ROWS_PER_SUB)

        csl = pl.ds(c * fh, fh)
        pltpu.async_copy(h_hbm.at[nsl, csl], h_st.at[nsl], sem0)
        pltpu.async_copy(h_hbm.at[nsl, csl], acc.at[nsl], sem0)
        pltpu.async_copy(src_hbm.at[pl.ds(base_row, rows_per_worker)], src_v,
                         sem0)
        pltpu.async_copy(dst_hbm.at[pl.ds(base_row, rows_per_worker)], dst_v,
                         sem0)
        pltpu.make_async_copy(h_hbm.at[nsl, csl], h_st.at[nsl], sem0).wait()
        pltpu.make_async_copy(h_hbm.at[nsl, csl], acc.at[nsl], sem0).wait()
        pltpu.make_async_copy(
            src_hbm.at[pl.ds(base_row, rows_per_worker)], src_v, sem0).wait()
        pltpu.make_async_copy(
            dst_hbm.at[pl.ds(base_row, rows_per_worker)], dst_v, sem0).wait()
        plsc.subcore_barrier()

        banks = ((rows0, semg0, sems0), (rows1, semg1, sems1))

        def gather_desc(g, rows_b, semg, j):
            return pltpu.make_async_copy(h_st.at[src_v.at[g * gsz + j]],
                                         rows_b.at[pl.ds(j * _K, _K)], semg)

        def scatter_desc(g, rows_b, sems, j):
            return pltpu.make_async_copy(rows_b.at[pl.ds(j * _K, _K)],
                                         acc.at[dst_v.at[g * gsz + j]], sems)

        def edge_pass():
            rows_p, semg_p, _ = banks[0]
            for j in range(gsz):
                gather_desc(0, rows_p, semg_p, j).start()

            @pl.loop(0, n_groups, step=2)
            def _(g0):
                for bi in range(2):
                    rows_c, semg_c, sems_c = banks[bi]
                    rows_n, semg_n, sems_n = banks[1 - bi]
                    g = g0 + bi

                    @pl.when(g >= 1)
                    def _():
                        for j in range(gsz):
                            scatter_desc(g - 1, rows_n, sems_n, j).wait()

                    @pl.when(g + 1 < n_groups)
                    def _():
                        for j in range(gsz):
                            gather_desc(g + 1, rows_n, semg_n, j).start()

                    for j in range(gsz):
                        gather_desc(g, rows_c, semg_c, j).wait()
                    for j in range(gsz):
                        scatter_desc(g, rows_c, sems_c, j).start(add=True)

            g_last = n_groups - 1
            rows_b, _, sems = banks[g_last % 2]
            for j in range(gsz):
                scatter_desc(g_last, rows_b, sems, j).wait()

        edge_pass()
        for _step in range(n_steps - 1):
            plsc.subcore_barrier()
            pieces = []
            off = 0
            while off < _ROWS_PER_SUB:
                sz = min(bank_rows, _ROWS_PER_SUB - off)
                pieces.append((off, sz, banks[len(pieces) % 2]))
                off += sz
            for off, sz, (rows_b, semg, _) in pieces:
                pltpu.async_copy(acc.at[pl.ds(row0 + off, sz)],
                                 rows_b.at[pl.ds(0, sz)], semg)
            for off, sz, (rows_b, semg, _) in pieces:
                pltpu.make_async_copy(acc.at[pl.ds(row0 + off, sz)],
                                      rows_b.at[pl.ds(0, sz)], semg).wait()
                pltpu.sync_copy(rows_b.at[pl.ds(0, sz)],
                                h_st.at[pl.ds(row0 + off, sz)])
            plsc.subcore_barrier()
            edge_pass()

        plsc.subcore_barrier()
        pltpu.sync_copy(acc.at[nsl], out_hbm.at[nsl, csl])

    return prop


def _mm1(x, w1, ei, e_pad):
    n = x.shape[0]
    fh = w1.shape[1] // 2
    e_rows = ei.shape[1] // _K
    pad_rows = e_pad // _K - e_rows

    def body(x_ref, w_ref, e_ref, o_ref, src_ref, dst_ref):
        y = jnp.dot(x_ref[...], w_ref[...], preferred_element_type=jnp.float32)
        zrows = jnp.zeros((_NP - n, 2 * fh), jnp.float32)
        o_ref[...] = jnp.concatenate([y, zrows], axis=0)
        src_ref[...] = jnp.concatenate(
            [jnp.reshape(e_ref[0], (e_rows, _K)),
             jnp.zeros((pad_rows, _K), jnp.int32)], axis=0)
        flat = (lax.broadcasted_iota(jnp.int32, (pad_rows, _K), 0) * _K
                + lax.broadcasted_iota(jnp.int32, (pad_rows, _K), 1))
        dump = _N + lax.rem(flat, jnp.int32(_NP - _N))
        dst_ref[...] = jnp.concatenate(
            [jnp.reshape(e_ref[1], (e_rows, _K)), dump], axis=0)

    return pl.pallas_call(
        body,
        out_shape=(
            jax.ShapeDtypeStruct((_NP, 2 * fh), jnp.float32),
            jax.ShapeDtypeStruct((e_pad // _K, _K), jnp.int32),
            jax.ShapeDtypeStruct((e_pad // _K, _K), jnp.int32),
        ),
    )(x, w1, ei)


def _mid(p, gamma, beta, w2):
    npad = p.shape[0]

    def body(p_ref, g_ref, b_ref, w_ref, o_ref):
        z = p_ref[...]
        rows = lax.broadcasted_iota(jnp.int32, (npad, 1), 0)
        mask = (rows < _N).astype(jnp.float32)
        zm = z * mask
        mean = jnp.sum(zm, axis=0, keepdims=True) * (1.0 / _N)
        d = (z - mean) * mask
        var = jnp.sum(d * d, axis=0, keepdims=True) * (1.0 / _N)
        hn = (z - mean) * jax.lax.rsqrt(var + _BN_EPS) * g_ref[...] + b_ref[...]
        hs = _SELU_SCALE * jnp.where(hn > 0, hn,
                                     _SELU_ALPHA * (jnp.exp(hn) - 1.0))
        o_ref[...] = jnp.dot(hs, w_ref[...], preferred_element_type=jnp.float32)

    return pl.pallas_call(
        body,
        out_shape=jax.ShapeDtypeStruct((npad, w2.shape[1]), jnp.float32),
    )(p, gamma, beta, w2)


def _final(p, b2, n):
    def body(p_ref, b_ref, o_ref):
        z = p_ref[:n] + b_ref[...]
        m = jnp.max(z, axis=1, keepdims=True)
        lse = jnp.log(jnp.sum(jnp.exp(z - m), axis=1, keepdims=True)) + m
        o_ref[...] = z - lse

    return pl.pallas_call(
        body,
        out_shape=jax.ShapeDtypeStruct((n, p.shape[1]), jnp.float32),
    )(p, b2)


@jax.jit
def kernel(x, edge_index, W1, b1, gamma, beta, W2, b2):
    del b1
    n, _ = x.shape
    e = edge_index.shape[1]
    quantum = _NS * _K * _G
    e_pad = ((e + quantum - 1) // quantum) * quantum

    ei = edge_index.astype(jnp.int32)

    prop_h = _make_prop(W1.shape[1] // 2, e_pad, 2, _G)
    prop_c = _make_prop(W2.shape[1] // 2, e_pad, 1, _G)

    y1, src_p, dst_p = _mm1(x, W1, ei, e_pad)
    p2 = prop_h(y1, src_p, dst_p)
    y2 = _mid(p2, gamma.reshape(1, -1), beta.reshape(1, -1), W2)
    p3 = prop_c(y2, src_p, dst_p)
    return _final(p3, b2.reshape(1, -1), n)
